# baseline (device time: 20251 ns/iter reference)
import jax
import jax.numpy as jnp
from jax import lax
from jax.experimental import pallas as pl
from jax.experimental.pallas import tpu as pltpu

MASKS = (1, 3, 4)


def kernel(t, W):
    m, k = t.shape
    _, n = W.shape
    n_rounds = len(MASKS)

    def body(t_ref, w_ref, out_ref, send_ref, recv_ref, send_sems, recv_sems):
        my_pos = lax.axis_index("i")

        barrier_sem = pltpu.get_barrier_semaphore()
        for mask in MASKS:
            pl.semaphore_signal(
                barrier_sem,
                inc=1,
                device_id=(my_pos ^ mask,),
                device_id_type=pl.DeviceIdType.MESH,
            )
        pl.semaphore_wait(barrier_sem, n_rounds)

        send_ref[0] = t_ref[...].astype(jnp.bfloat16)

        total = None
        for r, mask in enumerate(MASKS):
            rdma = pltpu.make_async_remote_copy(
                src_ref=send_ref.at[r],
                dst_ref=recv_ref.at[r],
                send_sem=send_sems.at[r],
                recv_sem=recv_sems.at[r],
                device_id=(my_pos ^ mask,),
                device_id_type=pl.DeviceIdType.MESH,
            )
            rdma.start()
            rdma.wait()
            total = send_ref[r] + recv_ref[r]
            if r + 1 < n_rounds:
                send_ref[r + 1] = total

        out_ref[...] = jnp.dot(
            total,
            w_ref[...].astype(jnp.bfloat16),
            preferred_element_type=jnp.float32,
        )

    return pl.pallas_call(
        body,
        out_shape=jax.ShapeDtypeStruct((m, n), jnp.float32),
        in_specs=[
            pl.BlockSpec(memory_space=pltpu.VMEM),
            pl.BlockSpec(memory_space=pltpu.VMEM),
        ],
        out_specs=pl.BlockSpec(memory_space=pltpu.VMEM),
        scratch_shapes=[
            pltpu.VMEM((n_rounds, m, k), jnp.bfloat16),
            pltpu.VMEM((n_rounds, m, k), jnp.bfloat16),
            pltpu.SemaphoreType.DMA((n_rounds,)),
            pltpu.SemaphoreType.DMA((n_rounds,)),
        ],
        compiler_params=pltpu.CompilerParams(collective_id=0),
    )(t, W)


# device time: 14545 ns/iter; 1.3923x vs baseline; 1.3923x over previous
import jax
import jax.numpy as jnp
from jax import lax
from jax.experimental import pallas as pl
from jax.experimental.pallas import tpu as pltpu

MASKS = (1, 3, 4)

SCHED = ((1, 3, 4), (3, 4, 1), (4, 1, 3))
GROUPS = ((0, 176), (176, 176), (352, 160))


def kernel(t, W):
    m, k = t.shape
    _, n = W.shape
    n_rounds = len(MASKS)

    def body(t_ref, w_ref, out_ref, send_ref, recv_ref, send_sems, recv_sems):
        my_pos = lax.axis_index("i")

        barrier_sem = pltpu.get_barrier_semaphore()
        for mask in MASKS:
            pl.semaphore_signal(
                barrier_sem,
                inc=1,
                device_id=(my_pos ^ mask,),
                device_id_type=pl.DeviceIdType.MESH,
            )
        pl.semaphore_wait(barrier_sem, n_rounds)

        send_ref[0] = t_ref[...].astype(jnp.bfloat16)

        total = None
        for r in range(n_rounds):
            rdmas = []
            for g, (off, length) in enumerate(GROUPS):
                rdma = pltpu.make_async_remote_copy(
                    src_ref=send_ref.at[r, pl.ds(off, length)],
                    dst_ref=recv_ref.at[r, pl.ds(off, length)],
                    send_sem=send_sems.at[r, g],
                    recv_sem=recv_sems.at[r, g],
                    device_id=(my_pos ^ SCHED[g][r],),
                    device_id_type=pl.DeviceIdType.MESH,
                )
                rdma.start()
                rdmas.append(rdma)
            for rdma in rdmas:
                rdma.wait()
            total = send_ref[r] + recv_ref[r]
            if r + 1 < n_rounds:
                send_ref[r + 1] = total

        out_ref[...] = jnp.dot(
            total,
            w_ref[...].astype(jnp.bfloat16),
            preferred_element_type=jnp.float32,
        )

    return pl.pallas_call(
        body,
        out_shape=jax.ShapeDtypeStruct((m, n), jnp.float32),
        in_specs=[
            pl.BlockSpec(memory_space=pltpu.VMEM),
            pl.BlockSpec(memory_space=pltpu.VMEM),
        ],
        out_specs=pl.BlockSpec(memory_space=pltpu.VMEM),
        scratch_shapes=[
            pltpu.VMEM((n_rounds, m, k), jnp.bfloat16),
            pltpu.VMEM((n_rounds, m, k), jnp.bfloat16),
            pltpu.SemaphoreType.DMA((n_rounds, len(GROUPS))),
            pltpu.SemaphoreType.DMA((n_rounds, len(GROUPS))),
        ],
        compiler_params=pltpu.CompilerParams(collective_id=0),
    )(t, W)
